# baseline (speedup 1.0000x reference)
import functools

import jax
import jax.numpy as jnp
from jax.experimental import pallas as pl

D_MODEL = 1024
D_HIDDEN = 4096
N_EXP = 8
TOP_K = 2

_RBLK = 512
_MBLK = 512
_HBLK = 512


def _router_body(x_ref, gw_ref, tw_ref, ti_ref, rk_ref, comb_ref, cnt_ref,
                 ps_ref):
    step = pl.program_id(0)
    blk = x_ref.shape[0]
    x = x_ref[...]
    gw = gw_ref[...]
    logits = jnp.dot(x, gw, preferred_element_type=jnp.float32)
    m = jnp.max(logits, axis=-1, keepdims=True)
    ex = jnp.exp(logits - m)
    probs = ex / jnp.sum(ex, axis=-1, keepdims=True)
    iota = jax.lax.broadcasted_iota(jnp.int32, probs.shape, 1)
    v1 = jnp.max(probs, axis=-1, keepdims=True)
    i1 = jnp.min(jnp.where(probs == v1, iota, N_EXP), axis=-1, keepdims=True)
    p2 = jnp.where(iota == i1, -jnp.inf, probs)
    v2 = jnp.max(p2, axis=-1, keepdims=True)
    i2 = jnp.min(jnp.where(p2 == v2, iota, N_EXP), axis=-1, keepdims=True)
    den = v1 + v2
    w1v = v1 / den
    w2v = v2 / den
    oh1 = (iota == i1)
    oh2 = (iota == i2)
    comb_ref[...] = (jnp.where(oh1, w1v, 0.0) + jnp.where(oh2, w2v, 0.0))
    tw_ref[...] = jnp.concatenate([w1v, w2v], axis=1)
    ti_ref[...] = jnp.concatenate([i1, i2], axis=1)

    @pl.when(step == 0)
    def _():
        cnt_ref[...] = jnp.zeros_like(cnt_ref)
        ps_ref[...] = jnp.zeros_like(ps_ref)

    pair = oh1.astype(jnp.int32) + oh2.astype(jnp.int32)
    c = pair
    sh = 1
    while sh < blk:
        c = c + jnp.concatenate(
            [jnp.zeros((sh, N_EXP), jnp.int32), c[:-sh]], axis=0)
        sh *= 2
    excl = c - pair
    base = cnt_ref[...]
    tot = base + excl
    rank0 = jnp.sum(jnp.where(oh1, tot, 0), axis=-1, keepdims=True)
    rank1 = jnp.sum(jnp.where(oh2, tot, 0), axis=-1, keepdims=True)
    rk_ref[...] = jnp.concatenate([rank0, rank1], axis=1)
    cnt_ref[...] = base + jnp.sum(pair, axis=0, keepdims=True)
    ps_ref[...] = ps_ref[...] + jnp.sum(probs, axis=0, keepdims=True)


def _run_router(x_flat, gate_w):
    T = x_flat.shape[0]
    grid = (T // _RBLK,)
    out_shape = [
        jax.ShapeDtypeStruct((T, TOP_K), jnp.float32),
        jax.ShapeDtypeStruct((T, TOP_K), jnp.int32),
        jax.ShapeDtypeStruct((T, TOP_K), jnp.int32),
        jax.ShapeDtypeStruct((T, N_EXP), jnp.float32),
        jax.ShapeDtypeStruct((1, N_EXP), jnp.int32),
        jax.ShapeDtypeStruct((1, N_EXP), jnp.float32),
    ]
    return pl.pallas_call(
        _router_body,
        grid=grid,
        in_specs=[
            pl.BlockSpec((_RBLK, D_MODEL), lambda i: (i, 0)),
            pl.BlockSpec((D_MODEL, N_EXP), lambda i: (0, 0)),
        ],
        out_specs=[
            pl.BlockSpec((_RBLK, TOP_K), lambda i: (i, 0)),
            pl.BlockSpec((_RBLK, TOP_K), lambda i: (i, 0)),
            pl.BlockSpec((_RBLK, TOP_K), lambda i: (i, 0)),
            pl.BlockSpec((_RBLK, N_EXP), lambda i: (i, 0)),
            pl.BlockSpec((1, N_EXP), lambda i: (0, 0)),
            pl.BlockSpec((1, N_EXP), lambda i: (0, 0)),
        ],
        out_shape=out_shape,
    )(x_flat, gate_w)


def _dense_ffn_body(x_ref, w1_ref, b1_ref, w2_ref, b2_ref, comb_ref, out_ref):
    e = pl.program_id(1)
    h = pl.program_id(2)

    @pl.when((e == 0) & (h == 0))
    def _():
        out_ref[...] = jnp.zeros_like(out_ref)

    xb = x_ref[...]
    hmat = jnp.dot(xb, w1_ref[0], preferred_element_type=jnp.float32)
    hmat = hmat + b1_ref[0]
    hmat = hmat * (1.0 / (1.0 + jnp.exp(-hmat)))
    acc = jnp.dot(hmat, w2_ref[0], preferred_element_type=jnp.float32)
    comb = comb_ref[...]
    eiota = jax.lax.broadcasted_iota(jnp.int32, comb.shape, 1)
    ce = jnp.sum(jnp.where(eiota == e, comb, 0.0), axis=1, keepdims=True)

    @pl.when(h == 0)
    def _():
        out_ref[...] += ce * b2_ref[0]

    out_ref[...] += ce * acc


def _run_dense_ffn(x_flat, w1, b1, w2, b2, comb):
    T = x_flat.shape[0]
    grid = (T // _MBLK, N_EXP, D_HIDDEN // _HBLK)
    b1r = b1.reshape(N_EXP, 1, D_HIDDEN)
    b2r = b2.reshape(N_EXP, 1, D_MODEL)
    return pl.pallas_call(
        _dense_ffn_body,
        grid=grid,
        in_specs=[
            pl.BlockSpec((_MBLK, D_MODEL), lambda m, e, h: (m, 0)),
            pl.BlockSpec((1, D_MODEL, _HBLK), lambda m, e, h: (e, 0, h)),
            pl.BlockSpec((1, 1, _HBLK), lambda m, e, h: (e, 0, h)),
            pl.BlockSpec((1, _HBLK, D_MODEL), lambda m, e, h: (e, h, 0)),
            pl.BlockSpec((1, 1, D_MODEL), lambda m, e, h: (e, 0, 0)),
            pl.BlockSpec((_MBLK, N_EXP), lambda m, e, h: (m, 0)),
        ],
        out_specs=pl.BlockSpec((_MBLK, D_MODEL), lambda m, e, h: (m, 0)),
        out_shape=jax.ShapeDtypeStruct((T, D_MODEL), jnp.float32),
    )(x_flat, w1, b1r, w2, b2r, comb)


def kernel(x, gate_w, w1, b1, w2, b2):
    B, S, D = x.shape
    x_flat = x.reshape(-1, D)
    T = x_flat.shape[0]
    tw, ti, rk, comb, cnt, ps = _run_router(x_flat, gate_w)
    out = _run_dense_ffn(x_flat, w1, b1, w2, b2, comb)
    f_i = cnt[0].astype(jnp.float32) / jnp.float32(T)
    p_i = ps[0] / jnp.float32(T)
    aux_loss = jnp.float32(N_EXP) * jnp.sum(f_i * p_i)
    return out.reshape(B, S, D), aux_loss
